# baseline (device time: 229656 ns/iter reference)
import jax
import jax.numpy as jnp
from jax import lax
from jax.experimental import pallas as pl
from jax.experimental.pallas import tpu as pltpu

B, S, HD_IN, HD_OUT = 4, 1024, 2048, 4096
S_HALF = S // 2
C = HD_OUT // 2


def kernel(O, Wo):
    O2 = O.reshape(B, S, HD_IN)
    Wo2 = Wo.astype(jnp.bfloat16)

    def body(
        o_hbm, wo_ref, out_hbm,
        land_rem, land_own, xsend, xrecv, yrecv, own, stage,
        xsend_sems, xrecv_sems, fsend_sems, yrecv_sems,
        rem_sem, own_sem, store_sem,
    ):
        my_x = lax.axis_index("x")
        my_y = lax.axis_index("y")
        x_nbr = (1 - my_x, my_y)
        y_nbr = (my_x, 1 - my_y)

        barrier_sem = pltpu.get_barrier_semaphore()
        for nbr in (x_nbr, y_nbr):
            pl.semaphore_signal(
                barrier_sem, inc=1,
                device_id=nbr, device_id_type=pl.DeviceIdType.MESH,
            )
        pl.semaphore_wait(barrier_sem, 2)

        own_rows = my_x * S_HALF
        rem_rows = (1 - my_x) * S_HALF

        def xrdma(b):
            return pltpu.make_async_remote_copy(
                src_ref=xsend.at[b % 2],
                dst_ref=xrecv.at[b],
                send_sem=xsend_sems.at[b % 2],
                recv_sem=xrecv_sems.at[b],
                device_id=x_nbr,
                device_id_type=pl.DeviceIdType.MESH,
            )

        def fwd(b):
            return pltpu.make_async_remote_copy(
                src_ref=xrecv.at[b],
                dst_ref=yrecv.at[b],
                send_sem=fsend_sems.at[b],
                recv_sem=yrecv_sems.at[b],
                device_id=y_nbr,
                device_id_type=pl.DeviceIdType.MESH,
            )

        def store(b):
            return pltpu.make_async_copy(stage, out_hbm.at[b], store_sem)

        def consume(b):
            fwd(b).wait_recv()
            if b > 0:
                store(b - 1).wait()

            @pl.when(my_y == 0)
            def _():
                stage[:, :C] = (
                    own[b % 2, :, :C].astype(jnp.float32)
                    + xrecv[b].astype(jnp.float32)
                ).astype(jnp.bfloat16)
                stage[:, C:] = (
                    own[b % 2, :, C:].astype(jnp.float32)
                    + yrecv[b].astype(jnp.float32)
                ).astype(jnp.bfloat16)

            @pl.when(my_y == 1)
            def _():
                stage[:, :C] = (
                    own[b % 2, :, :C].astype(jnp.float32)
                    + yrecv[b].astype(jnp.float32)
                ).astype(jnp.bfloat16)
                stage[:, C:] = (
                    own[b % 2, :, C:].astype(jnp.float32)
                    + xrecv[b].astype(jnp.float32)
                ).astype(jnp.bfloat16)

            store(b).start()

        for b in range(B):
            load_rem = pltpu.make_async_copy(
                o_hbm.at[b, pl.ds(rem_rows, S_HALF), :], land_rem, rem_sem
            )
            load_rem.start()
            load_own = pltpu.make_async_copy(
                o_hbm.at[b, pl.ds(own_rows, S_HALF), :], land_own, own_sem
            )
            load_own.start()

            if b >= 2:
                consume(b - 2)

            if b >= 2:
                xrdma(b - 2).wait_send()

            load_rem.wait()
            o_rem = land_rem[...].astype(jnp.bfloat16)

            @pl.when(my_y == 0)
            def _():
                xsend[b % 2] = jnp.dot(
                    o_rem, wo_ref[:, :C],
                    preferred_element_type=jnp.float32,
                ).astype(jnp.bfloat16)

            @pl.when(my_y == 1)
            def _():
                xsend[b % 2] = jnp.dot(
                    o_rem, wo_ref[:, C:],
                    preferred_element_type=jnp.float32,
                ).astype(jnp.bfloat16)

            xrdma(b).start()

            load_own.wait()
            o_own = land_own[...].astype(jnp.bfloat16)
            own[b % 2, :, :C] = jnp.dot(
                o_own, wo_ref[:, :C], preferred_element_type=jnp.float32
            ).astype(jnp.bfloat16)
            own[b % 2, :, C:] = jnp.dot(
                o_own, wo_ref[:, C:], preferred_element_type=jnp.float32
            ).astype(jnp.bfloat16)

            xrdma(b).wait_recv()
            fwd(b).start()

        consume(B - 2)
        consume(B - 1)
        for b in range(B - 2, B):
            xrdma(b).wait_send()
        for b in range(B):
            fwd(b).wait_send()
        store(B - 1).wait()

    return pl.pallas_call(
        body,
        out_shape=jax.ShapeDtypeStruct((B, S_HALF, HD_OUT), jnp.bfloat16),
        in_specs=[
            pl.BlockSpec(memory_space=pl.ANY),
            pl.BlockSpec(memory_space=pltpu.VMEM),
        ],
        out_specs=pl.BlockSpec(memory_space=pl.ANY),
        scratch_shapes=[
            pltpu.VMEM((S_HALF, HD_IN), jnp.float32),
            pltpu.VMEM((S_HALF, HD_IN), jnp.float32),
            pltpu.VMEM((2, S_HALF, C), jnp.bfloat16),
            pltpu.VMEM((B, S_HALF, C), jnp.bfloat16),
            pltpu.VMEM((B, S_HALF, C), jnp.bfloat16),
            pltpu.VMEM((2, S_HALF, HD_OUT), jnp.bfloat16),
            pltpu.VMEM((S_HALF, HD_OUT), jnp.bfloat16),
            pltpu.SemaphoreType.DMA((2,)),
            pltpu.SemaphoreType.DMA((B,)),
            pltpu.SemaphoreType.DMA((B,)),
            pltpu.SemaphoreType.DMA((B,)),
            pltpu.SemaphoreType.DMA,
            pltpu.SemaphoreType.DMA,
            pltpu.SemaphoreType.DMA,
        ],
        compiler_params=pltpu.CompilerParams(
            collective_id=0,
            vmem_limit_bytes=64 * 1024 * 1024,
        ),
    )(O2, Wo2)


# device time: 194751 ns/iter; 1.1792x vs baseline; 1.1792x over previous
import jax
import jax.numpy as jnp
from jax import lax
from jax.experimental import pallas as pl
from jax.experimental.pallas import tpu as pltpu

B, S, HD_IN, HD_OUT = 4, 1024, 2048, 4096
S_HALF = S // 2
C = HD_OUT // 2


def kernel(O, Wo):
    H, D = O.shape[2], O.shape[3]
    Wo2 = Wo.astype(jnp.bfloat16)

    def body(
        o_hbm, wo_ref, out_hbm,
        land_rem, land_own, xsend, xrecv, yrecv, own, stage,
        xsend_sems, xrecv_sems, fsend_sems, yrecv_sems,
        rem_sem, own_sem, store_sem,
    ):
        my_x = lax.axis_index("x")
        my_y = lax.axis_index("y")
        x_nbr = (1 - my_x, my_y)
        y_nbr = (my_x, 1 - my_y)

        barrier_sem = pltpu.get_barrier_semaphore()
        for nbr in (x_nbr, y_nbr):
            pl.semaphore_signal(
                barrier_sem, inc=1,
                device_id=nbr, device_id_type=pl.DeviceIdType.MESH,
            )
        pl.semaphore_wait(barrier_sem, 2)

        own_rows = my_x * S_HALF
        rem_rows = (1 - my_x) * S_HALF

        def xrdma(b):
            return pltpu.make_async_remote_copy(
                src_ref=xsend.at[b % 2],
                dst_ref=xrecv.at[b],
                send_sem=xsend_sems.at[b % 2],
                recv_sem=xrecv_sems.at[b],
                device_id=x_nbr,
                device_id_type=pl.DeviceIdType.MESH,
            )

        def fwd(b):
            return pltpu.make_async_remote_copy(
                src_ref=xrecv.at[b],
                dst_ref=yrecv.at[b],
                send_sem=fsend_sems.at[b],
                recv_sem=yrecv_sems.at[b],
                device_id=y_nbr,
                device_id_type=pl.DeviceIdType.MESH,
            )

        def store(b):
            return pltpu.make_async_copy(stage, out_hbm.at[b], store_sem)

        def consume(b):
            fwd(b).wait_recv()
            if b > 0:
                store(b - 1).wait()

            @pl.when(my_y == 0)
            def _():
                stage[:, :C] = (
                    own[b % 2, :, :C].astype(jnp.float32)
                    + xrecv[b].astype(jnp.float32)
                ).astype(jnp.bfloat16)
                stage[:, C:] = (
                    own[b % 2, :, C:].astype(jnp.float32)
                    + yrecv[b].astype(jnp.float32)
                ).astype(jnp.bfloat16)

            @pl.when(my_y == 1)
            def _():
                stage[:, :C] = (
                    own[b % 2, :, :C].astype(jnp.float32)
                    + yrecv[b].astype(jnp.float32)
                ).astype(jnp.bfloat16)
                stage[:, C:] = (
                    own[b % 2, :, C:].astype(jnp.float32)
                    + xrecv[b].astype(jnp.float32)
                ).astype(jnp.bfloat16)

            store(b).start()

        def head_gather(b, rows, land, sem):
            copies = [
                pltpu.make_async_copy(
                    o_hbm.at[b, pl.ds(rows, S_HALF), h, :],
                    land.at[:, pl.ds(h * D, D)],
                    sem,
                )
                for h in range(H)
            ]
            for c in copies:
                c.start()
            return copies

        for b in range(B):
            rem_copies = head_gather(b, rem_rows, land_rem, rem_sem)
            own_copies = head_gather(b, own_rows, land_own, own_sem)

            if b >= 2:
                consume(b - 2)

            if b >= 2:
                xrdma(b - 2).wait_send()

            for c in rem_copies:
                c.wait()
            o_rem = land_rem[...].astype(jnp.bfloat16)

            @pl.when(my_y == 0)
            def _():
                xsend[b % 2] = jnp.dot(
                    o_rem, wo_ref[:, :C],
                    preferred_element_type=jnp.float32,
                ).astype(jnp.bfloat16)

            @pl.when(my_y == 1)
            def _():
                xsend[b % 2] = jnp.dot(
                    o_rem, wo_ref[:, C:],
                    preferred_element_type=jnp.float32,
                ).astype(jnp.bfloat16)

            xrdma(b).start()

            for c in own_copies:
                c.wait()
            o_own = land_own[...].astype(jnp.bfloat16)
            own[b % 2, :, :C] = jnp.dot(
                o_own, wo_ref[:, :C], preferred_element_type=jnp.float32
            ).astype(jnp.bfloat16)
            own[b % 2, :, C:] = jnp.dot(
                o_own, wo_ref[:, C:], preferred_element_type=jnp.float32
            ).astype(jnp.bfloat16)

            xrdma(b).wait_recv()
            fwd(b).start()

        consume(B - 2)
        consume(B - 1)
        for b in range(B - 2, B):
            xrdma(b).wait_send()
        for b in range(B):
            fwd(b).wait_send()
        store(B - 1).wait()

    return pl.pallas_call(
        body,
        out_shape=jax.ShapeDtypeStruct((B, S_HALF, HD_OUT), jnp.bfloat16),
        in_specs=[
            pl.BlockSpec(memory_space=pl.ANY),
            pl.BlockSpec(memory_space=pltpu.VMEM),
        ],
        out_specs=pl.BlockSpec(memory_space=pl.ANY),
        scratch_shapes=[
            pltpu.VMEM((S_HALF, HD_IN), jnp.float32),
            pltpu.VMEM((S_HALF, HD_IN), jnp.float32),
            pltpu.VMEM((2, S_HALF, C), jnp.bfloat16),
            pltpu.VMEM((B, S_HALF, C), jnp.bfloat16),
            pltpu.VMEM((B, S_HALF, C), jnp.bfloat16),
            pltpu.VMEM((2, S_HALF, HD_OUT), jnp.bfloat16),
            pltpu.VMEM((S_HALF, HD_OUT), jnp.bfloat16),
            pltpu.SemaphoreType.DMA((2,)),
            pltpu.SemaphoreType.DMA((B,)),
            pltpu.SemaphoreType.DMA((B,)),
            pltpu.SemaphoreType.DMA((B,)),
            pltpu.SemaphoreType.DMA,
            pltpu.SemaphoreType.DMA,
            pltpu.SemaphoreType.DMA,
        ],
        compiler_params=pltpu.CompilerParams(
            collective_id=0,
            vmem_limit_bytes=64 * 1024 * 1024,
        ),
    )(O, Wo2)
